# Initial kernel scaffold; baseline (speedup 1.0000x reference)
import functools

import jax
import jax.numpy as jnp
from jax import lax
from jax.experimental import pallas as pl
from jax.experimental.pallas import tpu as pltpu
from jax.experimental.pallas import tpu_sc as plsc

NC, NS, LANES = 2, 16, 16
NW = NC * NS

WORD_DIM = 64
POS_DIM = 64
OUT_DIM = WORD_DIM + POS_DIM

CHUNK = 512
IDX_MINOR = 128
K = CHUNK // IDX_MINOR


def _emb_kernel(bt, word_hbm, pos_hbm, wtab_hbm, ptab_hbm, out_hbm,
                widx_v, pidx_v, wrows_v, prows_v, sem_idx, sem_rows):
    per_tile = bt // NW
    n_chunks = per_tile // CHUNK
    wid = lax.axis_index("s") * NC + lax.axis_index("c")
    tile_row0 = wid * (per_tile // IDX_MINOR)

    def body(g, carry):
        row0 = tile_row0 + g * K
        base = row0 * IDX_MINOR
        cp_w = pltpu.async_copy(word_hbm.at[pl.ds(row0, K)], widx_v, sem_idx)
        cp_p = pltpu.async_copy(pos_hbm.at[pl.ds(row0, K)], pidx_v, sem_idx)
        cp_w.wait()
        cp_p.wait()
        gathers = []
        for j in range(K):
            gathers.append(pltpu.async_copy(
                wtab_hbm.at[widx_v.at[j]],
                wrows_v.at[pl.ds(j * IDX_MINOR, IDX_MINOR)], sem_rows))
            gathers.append(pltpu.async_copy(
                ptab_hbm.at[pidx_v.at[j]],
                prows_v.at[pl.ds(j * IDX_MINOR, IDX_MINOR)], sem_rows))
        for cp in gathers:
            cp.wait()
        pltpu.sync_copy(wrows_v, out_hbm.at[pl.ds(base, CHUNK), pl.ds(0, WORD_DIM)])
        pltpu.sync_copy(prows_v, out_hbm.at[pl.ds(base, CHUNK), pl.ds(WORD_DIM, POS_DIM)])
        return carry

    lax.fori_loop(0, n_chunks, body, 0)


def kernel(word, pos, word_table, pos_table):
    b, l = word.shape
    bt = b * l
    word_flat = word.reshape(bt // IDX_MINOR, IDX_MINOR).astype(jnp.int32)
    pos_flat = pos.reshape(bt // IDX_MINOR, IDX_MINOR).astype(jnp.int32)

    mesh = plsc.VectorSubcoreMesh(core_axis_name="c", subcore_axis_name="s")
    out = pl.kernel(
        functools.partial(_emb_kernel, bt),
        out_type=jax.ShapeDtypeStruct((bt, OUT_DIM), jnp.float32),
        mesh=mesh,
        scratch_types=[
            pltpu.VMEM((K, IDX_MINOR), jnp.int32),
            pltpu.VMEM((K, IDX_MINOR), jnp.int32),
            pltpu.VMEM((CHUNK, WORD_DIM), jnp.float32),
            pltpu.VMEM((CHUNK, POS_DIM), jnp.float32),
            pltpu.SemaphoreType.DMA,
            pltpu.SemaphoreType.DMA,
        ],
    )(word_flat, pos_flat, word_table, pos_table)
    return out.reshape(b, l, OUT_DIM)


# speedup vs baseline: 3.4182x; 3.4182x over previous
import functools

import jax
import jax.numpy as jnp
from jax import lax
from jax.experimental import pallas as pl
from jax.experimental.pallas import tpu as pltpu
from jax.experimental.pallas import tpu_sc as plsc

NC, NS, LANES = 2, 16, 16
NW = NC * NS

WORD_DIM = 64
POS_DIM = 64
OUT_DIM = WORD_DIM + POS_DIM

CHUNK = 512
IDX_MINOR = 128
K = CHUNK // IDX_MINOR


def _emb_kernel(bt, word_hbm, pos_hbm, wtab_hbm, ptab_hbm, out_hbm,
                widx_v, pidx_v, wrows_v, prows_v, sem_idx, sem_rows):
    per_tile = bt // NW
    n_chunks = per_tile // CHUNK
    wid = lax.axis_index("s") * NC + lax.axis_index("c")
    tile_row0 = wid * (per_tile // IDX_MINOR)

    def body(g, carry):
        row0 = tile_row0 + g * K
        base = row0 * IDX_MINOR
        cp_w = pltpu.async_copy(word_hbm.at[pl.ds(row0, K)], widx_v, sem_idx)
        cp_p = pltpu.async_copy(pos_hbm.at[pl.ds(row0, K)], pidx_v, sem_idx)
        cp_w.wait()
        cp_p.wait()
        gathers = []
        for j in range(K):
            gathers.append(pltpu.async_copy(
                wtab_hbm.at[widx_v.at[j]],
                wrows_v.at[pl.ds(j * IDX_MINOR, IDX_MINOR)], sem_rows))
            gathers.append(pltpu.async_copy(
                ptab_hbm.at[pidx_v.at[j]],
                prows_v.at[pl.ds(j * IDX_MINOR, IDX_MINOR)], sem_rows))
        for cp in gathers:
            cp.wait()
        pltpu.sync_copy(wrows_v, out_hbm.at[pl.ds(base, CHUNK), pl.ds(0, WORD_DIM)])
        pltpu.sync_copy(prows_v, out_hbm.at[pl.ds(base, CHUNK), pl.ds(WORD_DIM, POS_DIM)])
        return carry

    lax.fori_loop(0, n_chunks, body, 0)


def kernel(word, pos, word_table, pos_table):
    b, l = word.shape
    bt = b * l
    word_flat = word.reshape(bt // IDX_MINOR, IDX_MINOR).astype(jnp.int32)
    pos_flat = pos.reshape(bt // IDX_MINOR, IDX_MINOR).astype(jnp.int32)

    mesh = plsc.VectorSubcoreMesh(core_axis_name="c", subcore_axis_name="s")
    out = pl.kernel(
        functools.partial(_emb_kernel, bt),
        out_type=jax.ShapeDtypeStruct((bt, OUT_DIM), jnp.float32),
        mesh=mesh,
        compiler_params=pltpu.CompilerParams(use_tc_tiling_on_sc=False),
        scratch_types=[
            pltpu.VMEM((K, IDX_MINOR), jnp.int32),
            pltpu.VMEM((K, IDX_MINOR), jnp.int32),
            pltpu.VMEM((CHUNK, WORD_DIM), jnp.float32),
            pltpu.VMEM((CHUNK, POS_DIM), jnp.float32),
            pltpu.SemaphoreType.DMA,
            pltpu.SemaphoreType.DMA,
        ],
    )(word_flat, pos_flat, word_table, pos_table)
    return out.reshape(b, l, OUT_DIM)


# speedup vs baseline: 3.4337x; 1.0045x over previous
import functools

import jax
import jax.numpy as jnp
from jax import lax
from jax.experimental import pallas as pl
from jax.experimental.pallas import tpu as pltpu
from jax.experimental.pallas import tpu_sc as plsc

NC, NS, LANES = 2, 16, 16
NW = NC * NS

WORD_DIM = 64
POS_DIM = 64
OUT_DIM = WORD_DIM + POS_DIM

IDX_MINOR = 128
K = 2
CHUNK = K * IDX_MINOR
NBUF = 2


def _emb_kernel(bt, word_hbm, pos_hbm, wtab_hbm, ptab_hbm, out_hbm,
                widx_v, pidx_v, wrows_v, prows_v,
                sem_idx, sem_rows, sem_wr):
    per_tile = bt // NW
    n_chunks = per_tile // CHUNK
    wid = lax.axis_index("s") * NC + lax.axis_index("c")
    tile_row0 = wid * (per_tile // IDX_MINOR)

    def idx_copies(g, b):
        row0 = tile_row0 + g * K
        return (
            pltpu.make_async_copy(word_hbm.at[pl.ds(row0, K)], widx_v.at[b],
                                  sem_idx.at[b]),
            pltpu.make_async_copy(pos_hbm.at[pl.ds(row0, K)], pidx_v.at[b],
                                  sem_idx.at[b]),
        )

    def write_copies(g, b):
        base = (tile_row0 + g * K) * IDX_MINOR
        return (
            pltpu.make_async_copy(
                wrows_v.at[b], out_hbm.at[pl.ds(base, CHUNK), pl.ds(0, WORD_DIM)],
                sem_wr.at[b]),
            pltpu.make_async_copy(
                prows_v.at[b],
                out_hbm.at[pl.ds(base, CHUNK), pl.ds(WORD_DIM, POS_DIM)],
                sem_wr.at[b]),
        )

    def fire_gathers(b):
        copies = []
        for j in range(K):
            copies.append(pltpu.make_async_copy(
                wtab_hbm.at[widx_v.at[b, j]],
                wrows_v.at[b, pl.ds(j * IDX_MINOR, IDX_MINOR)], sem_rows))
            copies.append(pltpu.make_async_copy(
                ptab_hbm.at[pidx_v.at[b, j]],
                prows_v.at[b, pl.ds(j * IDX_MINOR, IDX_MINOR)], sem_rows))
        for cp in copies:
            cp.start()
        return copies

    for b in range(NBUF):
        for cp in idx_copies(b, b):
            cp.start()

    def body(g2, carry):
        for b in range(NBUF):
            g = NBUF * g2 + b
            for cp in idx_copies(g, b):
                cp.wait()

            @pl.when(g2 > 0)
            def _():
                for cp in write_copies(g, b):
                    cp.wait()

            gathers = fire_gathers(b)
            for cp in gathers:
                cp.wait()

            @pl.when(g2 < n_chunks // NBUF - 1)
            def _():
                for cp in idx_copies(g + NBUF, b):
                    cp.start()

            for cp in write_copies(g, b):
                cp.start()
        return carry

    lax.fori_loop(0, n_chunks // NBUF, body, 0)

    for b in range(NBUF):
        g = n_chunks - NBUF + b
        for cp in write_copies(g, b):
            cp.wait()


def kernel(word, pos, word_table, pos_table):
    b, l = word.shape
    bt = b * l
    word_flat = word.reshape(bt // IDX_MINOR, IDX_MINOR).astype(jnp.int32)
    pos_flat = pos.reshape(bt // IDX_MINOR, IDX_MINOR).astype(jnp.int32)

    mesh = plsc.VectorSubcoreMesh(core_axis_name="c", subcore_axis_name="s")
    out = pl.kernel(
        functools.partial(_emb_kernel, bt),
        out_type=jax.ShapeDtypeStruct((bt, OUT_DIM), jnp.float32),
        mesh=mesh,
        compiler_params=pltpu.CompilerParams(use_tc_tiling_on_sc=False),
        scratch_types=[
            pltpu.VMEM((NBUF, K, IDX_MINOR), jnp.int32),
            pltpu.VMEM((NBUF, K, IDX_MINOR), jnp.int32),
            pltpu.VMEM((NBUF, CHUNK, WORD_DIM), jnp.float32),
            pltpu.VMEM((NBUF, CHUNK, POS_DIM), jnp.float32),
            pltpu.SemaphoreType.DMA((NBUF,)),
            pltpu.SemaphoreType.DMA,
            pltpu.SemaphoreType.DMA((NBUF,)),
        ],
    )(word_flat, pos_flat, word_table, pos_table)
    return out.reshape(b, l, OUT_DIM)


# speedup vs baseline: 4.2608x; 1.2409x over previous
import functools

import jax
import jax.numpy as jnp
from jax import lax
from jax.experimental import pallas as pl
from jax.experimental.pallas import tpu as pltpu
from jax.experimental.pallas import tpu_sc as plsc

NC, NS, LANES = 2, 16, 16
NW = NC * NS

WORD_DIM = 64
POS_DIM = 64
OUT_DIM = WORD_DIM + POS_DIM

IDX_MINOR = 128
K = 2
CHUNK = K * IDX_MINOR
NBUF = 2


def _emb_kernel(bt, word_hbm, pos_hbm, wtab_hbm, ptab_hbm, out_hbm,
                widx_v, pidx_v, wrows_v, prows_v,
                sem_idx, sem_rows, sem_wr):
    per_tile = bt // NW
    n_chunks = per_tile // CHUNK
    wid = lax.axis_index("s") * NC + lax.axis_index("c")
    tile_row0 = wid * (per_tile // IDX_MINOR)


    def idx_copies(g, b):
        row0 = tile_row0 + g * K
        return (
            pltpu.make_async_copy(word_hbm.at[pl.ds(row0, K)], widx_v.at[b],
                                  sem_idx.at[b]),
            pltpu.make_async_copy(pos_hbm.at[pl.ds(row0, K)], pidx_v.at[b],
                                  sem_idx.at[b]),
        )

    def write_copies(g, b):
        base = (tile_row0 + g * K) * IDX_MINOR
        return (
            pltpu.make_async_copy(
                wrows_v.at[b], out_hbm.at[pl.ds(base, CHUNK), pl.ds(0, WORD_DIM)],
                sem_wr.at[b]),
            pltpu.make_async_copy(
                prows_v.at[b],
                out_hbm.at[pl.ds(base, CHUNK), pl.ds(WORD_DIM, POS_DIM)],
                sem_wr.at[b]),
        )

    def fire_gathers(b):
        copies = []
        for j in range(K):
            copies.append(pltpu.make_async_copy(
                wtab_hbm.at[widx_v.at[b, j]],
                wrows_v.at[b, pl.ds(j * IDX_MINOR, IDX_MINOR)], sem_rows))
            copies.append(pltpu.make_async_copy(
                ptab_hbm.at[pidx_v.at[b, j]],
                prows_v.at[b, pl.ds(j * IDX_MINOR, IDX_MINOR)], sem_rows))
        for cp in copies:
            cp.start()
        return copies

    for b in range(NBUF):
        for cp in idx_copies(b, b):
            cp.start()

    def body(g2, carry):
        for b in range(NBUF):
            g = NBUF * g2 + b
            for cp in idx_copies(g, b):
                cp.wait()

            @pl.when(g2 > 0)
            def _():
                for cp in write_copies(g, b):
                    cp.wait()

            gathers = fire_gathers(b)
            for cp in gathers:
                cp.wait()

            @pl.when(g2 < n_chunks // NBUF - 1)
            def _():
                for cp in idx_copies(g + NBUF, b):
                    cp.start()

            for cp in write_copies(g, b):
                cp.start()
        return carry

    lax.fori_loop(0, n_chunks // NBUF, body, 0)

    for b in range(NBUF):
        g = n_chunks - NBUF + b
        for cp in write_copies(g, b):
            cp.wait()


def kernel(word, pos, word_table, pos_table):
    b, l = word.shape
    bt = b * l
    per_tile = bt // NW
    word_flat = word.reshape(bt // IDX_MINOR, IDX_MINOR).astype(jnp.int32)
    pos_size = pos_table.shape[0]
    ptab_rep = jnp.broadcast_to(
        pos_table[None], (NW,) + pos_table.shape).reshape(NW * pos_size,
                                                          pos_table.shape[1])
    rep_off = (jnp.arange(bt, dtype=jnp.int32) // per_tile) * pos_size
    pos_flat = (pos.reshape(bt).astype(jnp.int32) + rep_off).reshape(
        bt // IDX_MINOR, IDX_MINOR)

    mesh = plsc.VectorSubcoreMesh(core_axis_name="c", subcore_axis_name="s")
    out = pl.kernel(
        functools.partial(_emb_kernel, bt),
        out_type=jax.ShapeDtypeStruct((bt, OUT_DIM), jnp.float32),
        mesh=mesh,
        compiler_params=pltpu.CompilerParams(use_tc_tiling_on_sc=False),
        scratch_types=[
            pltpu.VMEM((NBUF, K, IDX_MINOR), jnp.int32),
            pltpu.VMEM((NBUF, K, IDX_MINOR), jnp.int32),
            pltpu.VMEM((NBUF, CHUNK, WORD_DIM), jnp.float32),
            pltpu.VMEM((NBUF, CHUNK, POS_DIM), jnp.float32),
            pltpu.SemaphoreType.DMA((NBUF,)),
            pltpu.SemaphoreType.DMA,
            pltpu.SemaphoreType.DMA((NBUF,)),
        ],
    )(word_flat, pos_flat, word_table, ptab_rep)
    return out.reshape(b, l, OUT_DIM)


# speedup vs baseline: 4.2729x; 1.0028x over previous
import functools

import jax
import jax.numpy as jnp
from jax import lax
from jax.experimental import pallas as pl
from jax.experimental.pallas import tpu as pltpu
from jax.experimental.pallas import tpu_sc as plsc

NC, NS, LANES = 2, 16, 16
NW = NC * NS

WORD_DIM = 64
POS_DIM = 64
OUT_DIM = WORD_DIM + POS_DIM

IDX_MINOR = 128
K = 2
CHUNK = K * IDX_MINOR
NBUF = 2


def _emb_kernel(bt, word_hbm, pos_hbm, wtab_hbm, ptab_hbm, out_hbm,
                widx_v, pidx_v, wrows_v, prows_v,
                sem_idx, sem_rows, sem_wr):
    per_tile = bt // NW
    n_chunks = per_tile // CHUNK
    wid = lax.axis_index("s") * NC + lax.axis_index("c")
    tile_row0 = wid * (per_tile // IDX_MINOR)


    def idx_copies(g, b):
        row0 = tile_row0 + g * K
        return (
            pltpu.make_async_copy(word_hbm.at[pl.ds(row0, K)], widx_v.at[b],
                                  sem_idx.at[b]),
            pltpu.make_async_copy(pos_hbm.at[pl.ds(row0, K)], pidx_v.at[b],
                                  sem_idx.at[b]),
        )

    def write_copies(g, b):
        base = (tile_row0 + g * K) * IDX_MINOR
        return (
            pltpu.make_async_copy(
                wrows_v.at[b], out_hbm.at[pl.ds(base, CHUNK), pl.ds(0, WORD_DIM)],
                sem_wr.at[b]),
            pltpu.make_async_copy(
                prows_v.at[b],
                out_hbm.at[pl.ds(base, CHUNK), pl.ds(WORD_DIM, POS_DIM)],
                sem_wr.at[b]),
        )

    def fire_gathers(b):
        copies = []
        for j in range(K):
            copies.append(pltpu.make_async_copy(
                wtab_hbm.at[widx_v.at[b, j]],
                wrows_v.at[b, pl.ds(j * IDX_MINOR, IDX_MINOR)], sem_rows))
            copies.append(pltpu.make_async_copy(
                ptab_hbm.at[pidx_v.at[b, j]],
                prows_v.at[b, pl.ds(j * IDX_MINOR, IDX_MINOR)], sem_rows))
        for cp in copies:
            cp.start()
        return copies

    for b in range(NBUF):
        for cp in idx_copies(b, b):
            cp.start()

    def body(g2, carry):
        for b in range(NBUF):
            g = NBUF * g2 + b
            for cp in idx_copies(g, b):
                cp.wait()

            @pl.when(g2 > 0)
            def _():
                for cp in write_copies(g, b):
                    cp.wait()

            gathers = fire_gathers(b)
            for cp in gathers:
                cp.wait()

            @pl.when(g2 < n_chunks // NBUF - 1)
            def _():
                for cp in idx_copies(g + NBUF, b):
                    cp.start()

            for cp in write_copies(g, b):
                cp.start()
        return carry

    lax.fori_loop(0, n_chunks // NBUF, body, 0)

    for b in range(NBUF):
        g = n_chunks - NBUF + b
        for cp in write_copies(g, b):
            cp.wait()


def kernel(word, pos, word_table, pos_table):
    b, l = word.shape
    bt = b * l
    per_tile = bt // NW
    word_flat = word.reshape(bt // IDX_MINOR, IDX_MINOR).astype(jnp.int32)
    pos_size = pos_table.shape[0]
    ptab_rep = jnp.broadcast_to(
        pos_table[None], (NW,) + pos_table.shape).reshape(NW * pos_size,
                                                          pos_table.shape[1])
    rows_per_worker = per_tile // l
    rep_off = (jnp.arange(b, dtype=jnp.int32) // rows_per_worker) * pos_size
    pos_flat = (pos.astype(jnp.int32) + rep_off[:, None]).reshape(
        bt // IDX_MINOR, IDX_MINOR)

    mesh = plsc.VectorSubcoreMesh(core_axis_name="c", subcore_axis_name="s")
    out = pl.kernel(
        functools.partial(_emb_kernel, bt),
        out_type=jax.ShapeDtypeStruct((bt, OUT_DIM), jnp.float32),
        mesh=mesh,
        compiler_params=pltpu.CompilerParams(use_tc_tiling_on_sc=False),
        scratch_types=[
            pltpu.VMEM((NBUF, K, IDX_MINOR), jnp.int32),
            pltpu.VMEM((NBUF, K, IDX_MINOR), jnp.int32),
            pltpu.VMEM((NBUF, CHUNK, WORD_DIM), jnp.float32),
            pltpu.VMEM((NBUF, CHUNK, POS_DIM), jnp.float32),
            pltpu.SemaphoreType.DMA((NBUF,)),
            pltpu.SemaphoreType.DMA,
            pltpu.SemaphoreType.DMA((NBUF,)),
        ],
    )(word_flat, pos_flat, word_table, ptab_rep)
    return out.reshape(b, l, OUT_DIM)


# speedup vs baseline: 4.5853x; 1.0731x over previous
import functools

import jax
import jax.numpy as jnp
from jax import lax
from jax.experimental import pallas as pl
from jax.experimental.pallas import tpu as pltpu
from jax.experimental.pallas import tpu_sc as plsc

NC, NS, LANES = 2, 16, 16
NW = NC * NS

WORD_DIM = 64
POS_DIM = 64
OUT_DIM = WORD_DIM + POS_DIM

IDX_MINOR = 128
K = 2
CHUNK = K * IDX_MINOR
NBUF = 2


def _emb_kernel(bt, word_hbm, pos_hbm, wtab_hbm, ptab_hbm, out_hbm,
                widx_v, pidx_v, wrows_v, prows_v,
                sem_idx, sem_rows, sem_wr):
    per_tile = bt // NW
    n_chunks = per_tile // CHUNK
    wid = lax.axis_index("s") * NC + lax.axis_index("c")
    tile_row0 = wid * (per_tile // IDX_MINOR)


    def idx_copies(g, b):
        row0 = tile_row0 + g * K
        return (
            pltpu.make_async_copy(word_hbm.at[pl.ds(row0, K)], widx_v.at[b],
                                  sem_idx.at[b]),
            pltpu.make_async_copy(pos_hbm.at[pl.ds(row0, K)], pidx_v.at[b],
                                  sem_idx.at[b]),
        )

    def write_copies(g, b):
        base = (tile_row0 + g * K) * IDX_MINOR
        return (
            pltpu.make_async_copy(
                wrows_v.at[b], out_hbm.at[pl.ds(base, CHUNK), pl.ds(0, WORD_DIM)],
                sem_wr.at[b]),
            pltpu.make_async_copy(
                prows_v.at[b],
                out_hbm.at[pl.ds(base, CHUNK), pl.ds(WORD_DIM, POS_DIM)],
                sem_wr.at[b]),
        )

    def fire_gathers(b):
        copies = []
        for j in range(K):
            copies.append(pltpu.make_async_copy(
                wtab_hbm.at[widx_v.at[b, j]],
                wrows_v.at[b, pl.ds(j * IDX_MINOR, IDX_MINOR)], sem_rows))
            copies.append(pltpu.make_async_copy(
                ptab_hbm.at[pidx_v.at[b, j]],
                prows_v.at[b, pl.ds(j * IDX_MINOR, IDX_MINOR)], sem_rows))
        for cp in copies:
            cp.start()
        return copies

    for b in range(NBUF):
        for cp in idx_copies(b, b):
            cp.start()

    def body(g2, carry):
        for b in range(NBUF):
            g = NBUF * g2 + b
            for cp in idx_copies(g, b):
                cp.wait()

            @pl.when(g2 > 0)
            def _():
                for cp in write_copies(g, b):
                    cp.wait()

            gathers = fire_gathers(b)
            for cp in gathers:
                cp.wait()

            @pl.when(g2 < n_chunks // NBUF - 1)
            def _():
                for cp in idx_copies(g + NBUF, b):
                    cp.start()

            for cp in write_copies(g, b):
                cp.start()
        return carry

    lax.fori_loop(0, n_chunks // NBUF, body, 0)

    for b in range(NBUF):
        g = n_chunks - NBUF + b
        for cp in write_copies(g, b):
            cp.wait()


def kernel(word, pos, word_table, pos_table):
    b, l = word.shape
    bt = b * l
    per_tile = bt // NW
    vocab = word_table.shape[0]
    wt_view = jnp.pad(word_table, ((0, 0), (0, IDX_MINOR - WORD_DIM))).reshape(
        2 * vocab, WORD_DIM)
    word_flat = (word.astype(jnp.int32) * 2).reshape(bt // IDX_MINOR, IDX_MINOR)
    pos_size = pos_table.shape[0]
    ptab_rep = jnp.broadcast_to(
        pos_table[None], (NW,) + pos_table.shape).reshape(NW * pos_size,
                                                          pos_table.shape[1])
    rows_per_worker = per_tile // l
    rep_off = (jnp.arange(b, dtype=jnp.int32) // rows_per_worker) * pos_size
    pos_flat = (pos.astype(jnp.int32) + rep_off[:, None]).reshape(
        bt // IDX_MINOR, IDX_MINOR)

    mesh = plsc.VectorSubcoreMesh(core_axis_name="c", subcore_axis_name="s")
    out = pl.kernel(
        functools.partial(_emb_kernel, bt),
        out_type=jax.ShapeDtypeStruct((bt, OUT_DIM), jnp.float32),
        mesh=mesh,
        compiler_params=pltpu.CompilerParams(use_tc_tiling_on_sc=False),
        scratch_types=[
            pltpu.VMEM((NBUF, K, IDX_MINOR), jnp.int32),
            pltpu.VMEM((NBUF, K, IDX_MINOR), jnp.int32),
            pltpu.VMEM((NBUF, CHUNK, WORD_DIM), jnp.float32),
            pltpu.VMEM((NBUF, CHUNK, POS_DIM), jnp.float32),
            pltpu.SemaphoreType.DMA((NBUF,)),
            pltpu.SemaphoreType.DMA,
            pltpu.SemaphoreType.DMA((NBUF,)),
        ],
    )(word_flat, pos_flat, wt_view, ptab_rep)
    return out.reshape(b, l, OUT_DIM)
